# baseline (speedup 1.0000x reference)
import functools

import jax
import jax.numpy as jnp
import numpy as np
from jax.experimental import pallas as pl
from jax.experimental.pallas import tpu as pltpu

B, N, F = 8, 128, 256
H, C = 3, 32
HC = H * C
K = int(np.ceil(0.6 * N))


def _gat_topk_kernel(x_ref, adj_ref, wl_ref, bl_ref, wr_ref, br_ref,
                     att_ref, we_ref, cb_ref, pw_ref, out_ref):
    xb = x_ref[0]
    ab = adj_ref[0]

    xl = jnp.dot(xb, wl_ref[...], preferred_element_type=jnp.float32) + bl_ref[0]
    xr = jnp.dot(xb, wr_ref[...], preferred_element_type=jnp.float32) + br_ref[0]

    cb = cb_ref[0]
    we = we_ref[0]
    att = att_ref[...]

    h_cols = []
    for h in range(H):
        xl_h = xl[:, h * C:(h + 1) * C]
        xr_h = xr[:, h * C:(h + 1) * C]
        we_h = we[h * C:(h + 1) * C]
        att_h = att[h]

        e = (xl_h[:, None, :] + xr_h[None, :, :]
             + ab[:, :, None] * we_h[None, None, :])
        e = jnp.where(e > 0, e, 0.2 * e)
        logit = jnp.sum(e * att_h[None, None, :], axis=-1)
        logit = jnp.where(ab != 0.0, logit, -1e9)

        m = jnp.max(logit, axis=0, keepdims=True)
        p = jnp.exp(logit - m)
        alpha = p / jnp.sum(p, axis=0, keepdims=True)

        out_h = jax.lax.dot_general(
            alpha, xl_h, (((0,), (0,)), ((), ())),
            preferred_element_type=jnp.float32)
        h_cols.append(out_h)

    hfull = jnp.concatenate(h_cols, axis=1) + cb[None, :]

    pw = pw_ref[0]
    inv_norm = jax.lax.rsqrt(jnp.sum(pw * pw))
    s = jnp.tanh(jnp.dot(hfull, pw[:, None],
                         preferred_element_type=jnp.float32) * inv_norm)
    sv = s[:, 0]
    gt = (sv[:, None] > sv[None, :]).astype(jnp.float32)
    idx = jax.lax.broadcasted_iota(jnp.int32, (N, N), 0)
    idy = jax.lax.broadcasted_iota(jnp.int32, (N, N), 1)
    eq = ((sv[:, None] == sv[None, :]) & (idx < idy)).astype(jnp.float32)
    rank = jnp.sum(gt + eq, axis=0)
    w = jnp.where(rank < float(K), sv, 0.0)
    out_ref[0, 0, :] = jnp.dot(w[None, :], hfull,
                               preferred_element_type=jnp.float32)[0] * (1.0 / K)


@jax.jit
def kernel(x, adj, Wl, bl, Wr, br, att, We, conv_bias, pool_w):
    out = pl.pallas_call(
        _gat_topk_kernel,
        grid=(B,),
        in_specs=[
            pl.BlockSpec((1, N, F), lambda b: (b, 0, 0)),
            pl.BlockSpec((1, N, N), lambda b: (b, 0, 0)),
            pl.BlockSpec((F, HC), lambda b: (0, 0)),
            pl.BlockSpec((1, HC), lambda b: (0, 0)),
            pl.BlockSpec((F, HC), lambda b: (0, 0)),
            pl.BlockSpec((1, HC), lambda b: (0, 0)),
            pl.BlockSpec((H, C), lambda b: (0, 0)),
            pl.BlockSpec((1, HC), lambda b: (0, 0)),
            pl.BlockSpec((1, HC), lambda b: (0, 0)),
            pl.BlockSpec((1, HC), lambda b: (0, 0)),
        ],
        out_specs=pl.BlockSpec((1, 1, HC), lambda b: (b, 0, 0)),
        out_shape=jax.ShapeDtypeStruct((B, 1, HC), jnp.float32),
        compiler_params=pltpu.CompilerParams(
            dimension_semantics=("arbitrary",)),
    )(x, adj, Wl, bl.reshape(1, HC), Wr, br.reshape(1, HC), att,
      We.reshape(1, HC), conv_bias.reshape(1, HC), pool_w.reshape(1, HC))
    return out[:, 0, :]


# speedup vs baseline: 2.4858x; 2.4858x over previous
import jax
import jax.numpy as jnp
import numpy as np
from jax.experimental import pallas as pl
from jax.experimental.pallas import tpu as pltpu

B, N, F = 8, 128, 256
H, C = 3, 32
HC = H * C
K = int(np.ceil(0.6 * N))


def _gat_topk_kernel(x_ref, adj_ref, wl_ref, bl_ref, wr_ref, br_ref,
                     att_ref, we_ref, cb_ref, pw_ref, out_ref):
    xb = x_ref[0]
    ab = adj_ref[0]
    att = att_ref[...]

    h_cols = []
    for h in range(H):
        wl_h = wl_ref[:, h * C:(h + 1) * C]
        wr_h = wr_ref[:, h * C:(h + 1) * C]
        bl_h = bl_ref[0, h * C:(h + 1) * C]
        br_h = br_ref[0, h * C:(h + 1) * C]
        we_h = we_ref[0, h * C:(h + 1) * C]
        att_h = att[h]
        w_abs = 0.4 * att_h
        w_lin = 0.6 * att_h

        xl_h = jnp.dot(xb, wl_h, preferred_element_type=jnp.float32) + bl_h[None, :]
        xrT = jax.lax.dot_general(wr_h, xb, (((0,), (1,)), ((), ())),
                                  preferred_element_type=jnp.float32) + br_h[:, None]

        al = jnp.dot(xl_h, w_lin[:, None],
                     preferred_element_type=jnp.float32)
        ar = jnp.dot(w_lin[None, :], xrT,
                     preferred_element_type=jnp.float32)
        aw = jnp.sum(w_lin * we_h)
        lin = al + ar + aw * ab

        sgn = jnp.where(w_abs >= 0.0, 1.0, -1.0)
        xlf = xl_h * w_abs[None, :]
        xrfT = xrT * w_abs[:, None]
        wef = we_h * w_abs
        zf = (xlf[:, :, None] + xrfT[None, :, :]
              + ab[:, None, :] * wef[None, :, None])
        asum = jnp.sum(jnp.abs(zf) * sgn[None, :, None], axis=1)

        logit = jnp.where(ab != 0.0, lin + asum, -1e9)

        m = jnp.max(logit, axis=0, keepdims=True)
        p = jnp.exp(logit - m)
        alpha = p / jnp.sum(p, axis=0, keepdims=True)

        out_h = jax.lax.dot_general(
            alpha, xl_h, (((0,), (0,)), ((), ())),
            preferred_element_type=jnp.float32)
        h_cols.append(out_h)

    hfull = jnp.concatenate(h_cols, axis=1) + cb_ref[0][None, :]

    pw = pw_ref[0]
    inv_norm = jax.lax.rsqrt(jnp.sum(pw * pw))
    s = jnp.tanh(jnp.dot(hfull, pw[:, None],
                         preferred_element_type=jnp.float32) * inv_norm)
    sv = s[:, 0]
    gt = (sv[:, None] > sv[None, :]).astype(jnp.float32)
    idx = jax.lax.broadcasted_iota(jnp.int32, (N, N), 0)
    idy = jax.lax.broadcasted_iota(jnp.int32, (N, N), 1)
    eq = ((sv[:, None] == sv[None, :]) & (idx < idy)).astype(jnp.float32)
    rank = jnp.sum(gt + eq, axis=0)
    w = jnp.where(rank < float(K), sv, 0.0)
    out_ref[0, 0, :] = jnp.dot(w[None, :], hfull,
                               preferred_element_type=jnp.float32)[0] * (1.0 / K)


@jax.jit
def kernel(x, adj, Wl, bl, Wr, br, att, We, conv_bias, pool_w):
    out = pl.pallas_call(
        _gat_topk_kernel,
        grid=(B,),
        in_specs=[
            pl.BlockSpec((1, N, F), lambda b: (b, 0, 0)),
            pl.BlockSpec((1, N, N), lambda b: (b, 0, 0)),
            pl.BlockSpec((F, HC), lambda b: (0, 0)),
            pl.BlockSpec((1, HC), lambda b: (0, 0)),
            pl.BlockSpec((F, HC), lambda b: (0, 0)),
            pl.BlockSpec((1, HC), lambda b: (0, 0)),
            pl.BlockSpec((H, C), lambda b: (0, 0)),
            pl.BlockSpec((1, HC), lambda b: (0, 0)),
            pl.BlockSpec((1, HC), lambda b: (0, 0)),
            pl.BlockSpec((1, HC), lambda b: (0, 0)),
        ],
        out_specs=pl.BlockSpec((1, 1, HC), lambda b: (b, 0, 0)),
        out_shape=jax.ShapeDtypeStruct((B, 1, HC), jnp.float32),
        compiler_params=pltpu.CompilerParams(
            dimension_semantics=("arbitrary",)),
    )(x, adj, Wl, bl.reshape(1, HC), Wr, br.reshape(1, HC), att,
      We.reshape(1, HC), conv_bias.reshape(1, HC), pool_w.reshape(1, HC))
    return out[:, 0, :]


# speedup vs baseline: 2.7355x; 1.1005x over previous
import jax
import jax.numpy as jnp
import numpy as np
from jax.experimental import pallas as pl
from jax.experimental.pallas import tpu as pltpu

B, N, F = 8, 128, 256
H, C = 3, 32
HC = H * C
K = int(np.ceil(0.6 * N))


def _gat_topk_kernel(x_ref, adj_ref, wl_ref, bl_ref, wr_ref, br_ref,
                     att_ref, we_ref, cb_ref, pw_ref, out_ref):
    xb = x_ref[0]
    ab = adj_ref[0]
    att = att_ref[...]

    h_cols = []
    for h in range(H):
        wl_h = wl_ref[:, h * C:(h + 1) * C]
        wr_h = wr_ref[:, h * C:(h + 1) * C]
        bl_h = bl_ref[0, h * C:(h + 1) * C]
        br_h = br_ref[0, h * C:(h + 1) * C]
        we_h = we_ref[0, h * C:(h + 1) * C]
        att_h = att[h]
        w_abs = 0.4 * att_h
        w_lin = 0.6 * att_h

        xl_h = jnp.dot(xb, wl_h, preferred_element_type=jnp.float32) + bl_h[None, :]
        xrT = jax.lax.dot_general(wr_h, xb, (((0,), (1,)), ((), ())),
                                  preferred_element_type=jnp.float32) + br_h[:, None]

        al = jnp.dot(xl_h, w_lin[:, None],
                     preferred_element_type=jnp.float32)
        ar = jnp.dot(w_lin[None, :], xrT,
                     preferred_element_type=jnp.float32)
        aw = jnp.sum(w_lin * we_h)
        lin = al + ar + aw * ab

        sgn = jnp.where(w_abs >= 0.0, 1.0, -1.0)
        xlf = xl_h * w_abs[None, :]
        xrfT = xrT * w_abs[:, None]
        wef = we_h * w_abs
        acc = lin
        for c in range(C):
            zc = xlf[:, c][:, None] + xrfT[c, :][None, :] + ab * wef[c]
            acc = acc + jnp.abs(zc) * sgn[c]

        logit = jnp.where(ab != 0.0, acc, -1e9)

        m = jnp.max(logit, axis=0, keepdims=True)
        p = jnp.exp(logit - m)
        alpha = p / jnp.sum(p, axis=0, keepdims=True)

        out_h = jax.lax.dot_general(
            alpha, xl_h, (((0,), (0,)), ((), ())),
            preferred_element_type=jnp.float32)
        h_cols.append(out_h)

    hfull = jnp.concatenate(h_cols, axis=1) + cb_ref[0][None, :]

    pw = pw_ref[0]
    inv_norm = jax.lax.rsqrt(jnp.sum(pw * pw))
    s = jnp.tanh(jnp.dot(hfull, pw[:, None],
                         preferred_element_type=jnp.float32) * inv_norm)
    sv = s[:, 0]
    gt = (sv[:, None] > sv[None, :]).astype(jnp.float32)
    idx = jax.lax.broadcasted_iota(jnp.int32, (N, N), 0)
    idy = jax.lax.broadcasted_iota(jnp.int32, (N, N), 1)
    eq = ((sv[:, None] == sv[None, :]) & (idx < idy)).astype(jnp.float32)
    rank = jnp.sum(gt + eq, axis=0)
    w = jnp.where(rank < float(K), sv, 0.0)
    out_ref[0, 0, :] = jnp.dot(w[None, :], hfull,
                               preferred_element_type=jnp.float32)[0] * (1.0 / K)


@jax.jit
def kernel(x, adj, Wl, bl, Wr, br, att, We, conv_bias, pool_w):
    out = pl.pallas_call(
        _gat_topk_kernel,
        grid=(B,),
        in_specs=[
            pl.BlockSpec((1, N, F), lambda b: (b, 0, 0)),
            pl.BlockSpec((1, N, N), lambda b: (b, 0, 0)),
            pl.BlockSpec((F, HC), lambda b: (0, 0)),
            pl.BlockSpec((1, HC), lambda b: (0, 0)),
            pl.BlockSpec((F, HC), lambda b: (0, 0)),
            pl.BlockSpec((1, HC), lambda b: (0, 0)),
            pl.BlockSpec((H, C), lambda b: (0, 0)),
            pl.BlockSpec((1, HC), lambda b: (0, 0)),
            pl.BlockSpec((1, HC), lambda b: (0, 0)),
            pl.BlockSpec((1, HC), lambda b: (0, 0)),
        ],
        out_specs=pl.BlockSpec((1, 1, HC), lambda b: (b, 0, 0)),
        out_shape=jax.ShapeDtypeStruct((B, 1, HC), jnp.float32),
        compiler_params=pltpu.CompilerParams(
            dimension_semantics=("arbitrary",)),
    )(x, adj, Wl, bl.reshape(1, HC), Wr, br.reshape(1, HC), att,
      We.reshape(1, HC), conv_bias.reshape(1, HC), pool_w.reshape(1, HC))
    return out[:, 0, :]


# speedup vs baseline: 3.8935x; 1.4233x over previous
import jax
import jax.numpy as jnp
import numpy as np
from jax.experimental import pallas as pl
from jax.experimental.pallas import tpu as pltpu

B, N, F = 8, 128, 256
H, C = 3, 32
HC = H * C
K = int(np.ceil(0.6 * N))


def _gat_topk_kernel(x_ref, adj_ref, wl_ref, bl_ref, wr_ref, br_ref,
                     att3_ref, we_ref, cb_ref, pw_ref, out_ref):
    xb = x_ref[0]
    ab = adj_ref[0]

    xlT = jax.lax.dot_general(wl_ref[...], xb, (((0,), (1,)), ((), ())),
                              preferred_element_type=jnp.float32) + bl_ref[0][:, None]
    xrT = jax.lax.dot_general(wr_ref[...], xb, (((0,), (1,)), ((), ())),
                              preferred_element_type=jnp.float32) + br_ref[0][:, None]

    we = we_ref[0]
    e3 = (xlT.T[:, :, None] + xrT[None, :, :]
          + ab[:, None, :] * we[None, :, None])
    e3 = jnp.where(e3 > 0, e3, 0.2 * e3)
    att3b = jnp.broadcast_to(att3_ref[...][None], (N, H, HC))
    logits = jax.lax.dot_general(att3b, e3, (((2,), (1,)), ((0,), (0,))),
                                 preferred_element_type=jnp.float32)

    mask3 = (ab != 0.0)[:, None, :]
    logits = jnp.where(mask3, logits, -1e9)
    m = jnp.max(logits, axis=0, keepdims=True)
    p = jnp.exp(logits - m)
    den = jnp.sum(p, axis=0)

    outall = jax.lax.dot_general(xlT, p, (((1,), (0,)), ((), ())),
                                 preferred_element_type=jnp.float32)
    h_rows = [outall[h * C:(h + 1) * C, h, :] / den[h][None, :] for h in range(H)]
    hfullT = jnp.concatenate(h_rows, axis=0) + cb_ref[0][:, None]

    pw = pw_ref[0]
    inv_norm = jax.lax.rsqrt(jnp.sum(pw * pw))
    s = jnp.tanh(jax.lax.dot_general(pw[None, :], hfullT,
                                     (((1,), (0,)), ((), ())),
                                     preferred_element_type=jnp.float32) * inv_norm)
    sv = s[0]
    gt = (sv[:, None] > sv[None, :]).astype(jnp.float32)
    idx = jax.lax.broadcasted_iota(jnp.int32, (N, N), 0)
    idy = jax.lax.broadcasted_iota(jnp.int32, (N, N), 1)
    eq = ((sv[:, None] == sv[None, :]) & (idx < idy)).astype(jnp.float32)
    rank = jnp.sum(gt + eq, axis=0)
    w = jnp.where(rank < float(K), sv, 0.0)
    out_ref[0, 0, :] = jax.lax.dot_general(
        hfullT, w[:, None], (((1,), (0,)), ((), ())),
        preferred_element_type=jnp.float32)[:, 0] * (1.0 / K)


@jax.jit
def kernel(x, adj, Wl, bl, Wr, br, att, We, conv_bias, pool_w):
    att3 = (jnp.eye(H, dtype=jnp.float32)[:, :, None] * att[None, :, :]).reshape(H, HC)
    out = pl.pallas_call(
        _gat_topk_kernel,
        grid=(B,),
        in_specs=[
            pl.BlockSpec((1, N, F), lambda b: (b, 0, 0)),
            pl.BlockSpec((1, N, N), lambda b: (b, 0, 0)),
            pl.BlockSpec((F, HC), lambda b: (0, 0)),
            pl.BlockSpec((1, HC), lambda b: (0, 0)),
            pl.BlockSpec((F, HC), lambda b: (0, 0)),
            pl.BlockSpec((1, HC), lambda b: (0, 0)),
            pl.BlockSpec((H, HC), lambda b: (0, 0)),
            pl.BlockSpec((1, HC), lambda b: (0, 0)),
            pl.BlockSpec((1, HC), lambda b: (0, 0)),
            pl.BlockSpec((1, HC), lambda b: (0, 0)),
        ],
        out_specs=pl.BlockSpec((1, 1, HC), lambda b: (b, 0, 0)),
        out_shape=jax.ShapeDtypeStruct((B, 1, HC), jnp.float32),
        compiler_params=pltpu.CompilerParams(
            dimension_semantics=("arbitrary",)),
    )(x, adj, Wl, bl.reshape(1, HC), Wr, br.reshape(1, HC), att3,
      We.reshape(1, HC), conv_bias.reshape(1, HC), pool_w.reshape(1, HC))
    return out[:, 0, :]


# speedup vs baseline: 4.0718x; 1.0458x over previous
import jax
import jax.numpy as jnp
import numpy as np
from jax.experimental import pallas as pl
from jax.experimental.pallas import tpu as pltpu

B, N, F = 8, 128, 256
H, C = 3, 32
HC = H * C
K = int(np.ceil(0.6 * N))
PAIR = 2


def _one_graph(xb, ab, wl, bl, wr, br, att3, we, cb, pw):
    xlT = jax.lax.dot_general(wl, xb, (((0,), (1,)), ((), ())),
                              preferred_element_type=jnp.float32) + bl[:, None]
    xrT = jax.lax.dot_general(wr, xb, (((0,), (1,)), ((), ())),
                              preferred_element_type=jnp.float32) + br[:, None]

    e3 = (xlT.T[:, :, None] + xrT[None, :, :]
          + ab[:, None, :] * we[None, :, None])
    e3 = jnp.maximum(e3, 0.2 * e3)
    att3b = jnp.broadcast_to(att3[None], (N, H, HC))
    logits = jax.lax.dot_general(att3b, e3, (((2,), (1,)), ((0,), (0,))),
                                 preferred_element_type=jnp.float32)

    mask3 = (ab != 0.0)[:, None, :]
    logits = jnp.where(mask3, logits, -1e9)
    m = jnp.max(logits, axis=0, keepdims=True)
    p = jnp.exp(logits - m)
    den = jnp.sum(p, axis=0)

    outall = jax.lax.dot_general(xlT, p, (((1,), (0,)), ((), ())),
                                 preferred_element_type=jnp.float32)
    h_rows = [outall[h * C:(h + 1) * C, h, :] / den[h][None, :] for h in range(H)]
    hfullT = jnp.concatenate(h_rows, axis=0) + cb[:, None]

    inv_norm = jax.lax.rsqrt(jnp.sum(pw * pw))
    s = jnp.tanh(jax.lax.dot_general(pw[None, :], hfullT,
                                     (((1,), (0,)), ((), ())),
                                     preferred_element_type=jnp.float32) * inv_norm)
    sv = s[0]
    gt = (sv[:, None] > sv[None, :]).astype(jnp.float32)
    idx = jax.lax.broadcasted_iota(jnp.int32, (N, N), 0)
    idy = jax.lax.broadcasted_iota(jnp.int32, (N, N), 1)
    eq = ((sv[:, None] == sv[None, :]) & (idx < idy)).astype(jnp.float32)
    rank = jnp.sum(gt + eq, axis=0)
    w = jnp.where(rank < float(K), sv, 0.0)
    return jax.lax.dot_general(
        hfullT, w[:, None], (((1,), (0,)), ((), ())),
        preferred_element_type=jnp.float32)[:, 0] * (1.0 / K)


def _gat_topk_kernel(x_ref, adj_ref, wl_ref, bl_ref, wr_ref, br_ref,
                     att3_ref, we_ref, cb_ref, pw_ref, out_ref):
    wl = wl_ref[...]
    wr = wr_ref[...]
    att3 = att3_ref[...]
    bl, br, we, cb, pw = (bl_ref[0], br_ref[0], we_ref[0], cb_ref[0],
                          pw_ref[0])
    for g in range(PAIR):
        out_ref[g, 0, :] = _one_graph(x_ref[g], adj_ref[g], wl, bl, wr, br,
                                      att3, we, cb, pw)


@jax.jit
def kernel(x, adj, Wl, bl, Wr, br, att, We, conv_bias, pool_w):
    att3 = (jnp.eye(H, dtype=jnp.float32)[:, :, None] * att[None, :, :]).reshape(H, HC)
    out = pl.pallas_call(
        _gat_topk_kernel,
        grid=(B // PAIR,),
        in_specs=[
            pl.BlockSpec((PAIR, N, F), lambda b: (b, 0, 0)),
            pl.BlockSpec((PAIR, N, N), lambda b: (b, 0, 0)),
            pl.BlockSpec((F, HC), lambda b: (0, 0)),
            pl.BlockSpec((1, HC), lambda b: (0, 0)),
            pl.BlockSpec((F, HC), lambda b: (0, 0)),
            pl.BlockSpec((1, HC), lambda b: (0, 0)),
            pl.BlockSpec((H, HC), lambda b: (0, 0)),
            pl.BlockSpec((1, HC), lambda b: (0, 0)),
            pl.BlockSpec((1, HC), lambda b: (0, 0)),
            pl.BlockSpec((1, HC), lambda b: (0, 0)),
        ],
        out_specs=pl.BlockSpec((PAIR, 1, HC), lambda b: (b, 0, 0)),
        out_shape=jax.ShapeDtypeStruct((B, 1, HC), jnp.float32),
        compiler_params=pltpu.CompilerParams(
            dimension_semantics=("arbitrary",)),
    )(x, adj, Wl, bl.reshape(1, HC), Wr, br.reshape(1, HC), att3,
      We.reshape(1, HC), conv_bias.reshape(1, HC), pool_w.reshape(1, HC))
    return out[:, 0, :]


# speedup vs baseline: 4.1767x; 1.0258x over previous
import jax
import jax.numpy as jnp
import numpy as np
from jax.experimental import pallas as pl
from jax.experimental.pallas import tpu as pltpu

B, N, F = 8, 128, 256
H, C = 3, 32
HC = H * C
K = int(np.ceil(0.6 * N))
PAIR = 4


def _one_graph(xb, ab, wl, bl, wr, br, att3, we, cb, pw):
    xlT = jax.lax.dot_general(wl, xb, (((0,), (1,)), ((), ())),
                              preferred_element_type=jnp.float32) + bl[:, None]
    xrT = jax.lax.dot_general(wr, xb, (((0,), (1,)), ((), ())),
                              preferred_element_type=jnp.float32) + br[:, None]

    e3 = (xlT.T[:, :, None] + xrT[None, :, :]
          + ab[:, None, :] * we[None, :, None])
    e3 = jnp.maximum(e3, 0.2 * e3)
    att3b = jnp.broadcast_to(att3[None], (N, H, HC))
    logits = jax.lax.dot_general(att3b, e3, (((2,), (1,)), ((0,), (0,))),
                                 preferred_element_type=jnp.float32)

    mask3 = (ab != 0.0)[:, None, :]
    logits = jnp.where(mask3, logits, -1e9)
    m = jnp.max(logits, axis=0, keepdims=True)
    p = jnp.exp(logits - m)
    den = jnp.sum(p, axis=0)

    outall = jax.lax.dot_general(xlT, p, (((1,), (0,)), ((), ())),
                                 preferred_element_type=jnp.float32)
    h_rows = [outall[h * C:(h + 1) * C, h, :] / den[h][None, :] for h in range(H)]
    hfullT = jnp.concatenate(h_rows, axis=0) + cb[:, None]

    inv_norm = jax.lax.rsqrt(jnp.sum(pw * pw))
    s = jnp.tanh(jax.lax.dot_general(pw[None, :], hfullT,
                                     (((1,), (0,)), ((), ())),
                                     preferred_element_type=jnp.float32) * inv_norm)
    sv = s[0]
    gt = (sv[:, None] > sv[None, :]).astype(jnp.float32)
    idx = jax.lax.broadcasted_iota(jnp.int32, (N, N), 0)
    idy = jax.lax.broadcasted_iota(jnp.int32, (N, N), 1)
    eq = ((sv[:, None] == sv[None, :]) & (idx < idy)).astype(jnp.float32)
    rank = jnp.sum(gt + eq, axis=0)
    w = jnp.where(rank < float(K), sv, 0.0)
    return jax.lax.dot_general(
        hfullT, w[:, None], (((1,), (0,)), ((), ())),
        preferred_element_type=jnp.float32)[:, 0] * (1.0 / K)


def _gat_topk_kernel(x_ref, adj_ref, wl_ref, bl_ref, wr_ref, br_ref,
                     att3_ref, we_ref, cb_ref, pw_ref, out_ref):
    wl = wl_ref[...]
    wr = wr_ref[...]
    att3 = att3_ref[...]
    bl, br, we, cb, pw = (bl_ref[0], br_ref[0], we_ref[0], cb_ref[0],
                          pw_ref[0])
    for g in range(PAIR):
        out_ref[g, 0, :] = _one_graph(x_ref[g], adj_ref[g], wl, bl, wr, br,
                                      att3, we, cb, pw)


@jax.jit
def kernel(x, adj, Wl, bl, Wr, br, att, We, conv_bias, pool_w):
    att3 = (jnp.eye(H, dtype=jnp.float32)[:, :, None] * att[None, :, :]).reshape(H, HC)
    out = pl.pallas_call(
        _gat_topk_kernel,
        grid=(B // PAIR,),
        in_specs=[
            pl.BlockSpec((PAIR, N, F), lambda b: (b, 0, 0)),
            pl.BlockSpec((PAIR, N, N), lambda b: (b, 0, 0)),
            pl.BlockSpec((F, HC), lambda b: (0, 0)),
            pl.BlockSpec((1, HC), lambda b: (0, 0)),
            pl.BlockSpec((F, HC), lambda b: (0, 0)),
            pl.BlockSpec((1, HC), lambda b: (0, 0)),
            pl.BlockSpec((H, HC), lambda b: (0, 0)),
            pl.BlockSpec((1, HC), lambda b: (0, 0)),
            pl.BlockSpec((1, HC), lambda b: (0, 0)),
            pl.BlockSpec((1, HC), lambda b: (0, 0)),
        ],
        out_specs=pl.BlockSpec((PAIR, 1, HC), lambda b: (b, 0, 0)),
        out_shape=jax.ShapeDtypeStruct((B, 1, HC), jnp.float32),
        compiler_params=pltpu.CompilerParams(
            dimension_semantics=("arbitrary",)),
    )(x, adj, Wl, bl.reshape(1, HC), Wr, br.reshape(1, HC), att3,
      We.reshape(1, HC), conv_bias.reshape(1, HC), pool_w.reshape(1, HC))
    return out[:, 0, :]
